# Initial kernel scaffold; baseline (speedup 1.0000x reference)
import jax
import jax.numpy as jnp
from jax.experimental import pallas as pl


def kernel(x, edge_attr, enc_W, enc_b, bond_W, bond_b, eps, mlp_W1, mlp_b1, mlp_g1, mlp_be1, mlp_W2, mlp_b2, bn_g, bn_b, vn_W1, vn_b1, vn_g1, vn_be1, vn_W2, vn_b2, vn_g2, vn_be2, head_W, head_b, edge_index, batch):
    raise NotImplementedError("write your pallas kernel here")


# speedup vs baseline: 1.5218x; 1.5218x over previous
import functools

import jax
import jax.numpy as jnp
from jax import lax
from jax.experimental import pallas as pl
from jax.experimental.pallas import tpu as pltpu
from jax.experimental.pallas import tpu_sc as plsc

N = 10000
E = 320000
D = 128
De = 16
L = 5
G = 64
T = 128

NC = 2
NS = 16
NW = NC * NS
CH = 128
CPT = 80
EPAD = NW * CPT * CH
IDX_ROWS = EPAD // CH
NP = N + 112
RPT = NP // NS
IHALF = CPT // 2

_F32 = jnp.float32


def _edge_body(hin, ee, srcr, dstr, out, acc, src_v, dst_v, ee_v, h_v, sem):
    c = lax.axis_index("c")
    s = lax.axis_index("s")
    wid = s * NC + c

    def _zrow(i, _):
        for dd in range(D // 16):
            h_v[i, pl.ds(dd * 16, 16)] = jnp.zeros((16,), _F32)
        return 0

    lax.fori_loop(0, CH, _zrow, 0)
    r0 = s * RPT
    off = 0
    while off < RPT:
        nrows = min(CH, RPT - off)
        pltpu.sync_copy(h_v.at[pl.ds(0, nrows)], acc.at[pl.ds(r0 + off, nrows)])
        off += nrows
    plsc.subcore_barrier()

    for half in range(CPT // IHALF):
        ibase = wid * CPT + half * IHALF
        pltpu.sync_copy(srcr.at[pl.ds(ibase, IHALF)], src_v)
        pltpu.sync_copy(dstr.at[pl.ds(ibase, IHALF)], dst_v)

        def _chunk(j, _):
            base = (ibase + j) * CH
            pltpu.sync_copy(ee.at[pl.ds(base, CH)], ee_v)
            pltpu.async_copy(hin.at[src_v.at[j]], h_v, sem).wait()

            def _crow(i, _):
                for dd in range(D // 16):
                    sl = pl.ds(dd * 16, 16)
                    h_v[i, sl] = jnp.maximum(h_v[i, sl] + ee_v[i, sl], 0.0)
                return 0

            lax.fori_loop(0, CH, _crow, 0)
            pltpu.sync_copy(h_v, acc.at[dst_v.at[j]], add=True)
            return 0

        lax.fori_loop(0, IHALF, _chunk, 0)
    plsc.subcore_barrier()

    pltpu.sync_copy(acc.at[pl.ds(s * RPT, RPT)],
                    out.at[pl.ds(c * NP + s * RPT, RPT)])


_edge_kernel = functools.partial(
    pl.kernel,
    out_type=jax.ShapeDtypeStruct((2 * NP, D), _F32),
    mesh=plsc.VectorSubcoreMesh(core_axis_name="c", subcore_axis_name="s"),
    scratch_types=[
        pltpu.VMEM_SHARED((NP, D), _F32),
        pltpu.VMEM((IHALF, CH), jnp.int32),
        pltpu.VMEM((IHALF, CH), jnp.int32),
        pltpu.VMEM((CH, D), _F32),
        pltpu.VMEM((CH, D), _F32),
        pltpu.SemaphoreType.DMA,
    ],
)(_edge_body)


def _dot(a, b):
    return jnp.dot(a.astype(jnp.bfloat16), b.astype(jnp.bfloat16),
                   preferred_element_type=_F32)


def _bn_stats_ref(ref, n):
    t = n // 8
    bb = t // 2
    cc = ref.shape[1]

    def mstep(k, accs):
        a0, a1 = accs
        return (a0 + ref[pl.ds(8 * k, 8), :], a1 + ref[pl.ds(8 * (bb + k), 8), :])

    a0, a1 = lax.fori_loop(0, bb, mstep,
                           (jnp.zeros((8, cc), _F32), jnp.zeros((8, cc), _F32)))
    m = jnp.sum(a0 + a1, axis=0, keepdims=True) / float(n)

    def vstep(k, accs):
        a0, a1 = accs
        d0 = ref[pl.ds(8 * k, 8), :] - m
        d1 = ref[pl.ds(8 * (bb + k), 8), :] - m
        return (a0 + d0 * d0, a1 + d1 * d1)

    a0, a1 = lax.fori_loop(0, bb, vstep,
                           (jnp.zeros((8, cc), _F32), jnp.zeros((8, cc), _F32)))
    v = jnp.sum(a0 + a1, axis=0, keepdims=True) / float(n)
    return m, v


def _bn_ref(ref, n, g, b):
    m, v = _bn_stats_ref(ref, n)
    return g * (ref[...] - m) / jnp.sqrt(v + 1e-5) + b


def _bn_small(h, n, g, b):
    t = n // 8
    bb = t // 2
    cc = h.shape[1]
    a0 = jnp.zeros((8, cc), _F32)
    a1 = jnp.zeros((8, cc), _F32)
    for k in range(bb):
        a0 = a0 + h[8 * k:8 * k + 8, :]
        a1 = a1 + h[8 * (bb + k):8 * (bb + k) + 8, :]
    m = jnp.sum(a0 + a1, axis=0, keepdims=True) / float(n)
    a0 = jnp.zeros((8, cc), _F32)
    a1 = jnp.zeros((8, cc), _F32)
    for k in range(bb):
        d0 = h[8 * k:8 * k + 8, :] - m
        d1 = h[8 * (bb + k):8 * (bb + k) + 8, :] - m
        a0 = a0 + d0 * d0
        a1 = a1 + d1 * d1
    v = jnp.sum(a0 + a1, axis=0, keepdims=True) / float(n)
    return g * (h - m) / jnp.sqrt(v + 1e-5) + b


def _enc_body(x_ref, w_ref, b_ref, bcol_ref, brow_ref, h_ref, oh_ref, oht_ref):
    h_ref[...] = _dot(x_ref[...], w_ref[...]) + b_ref[...]
    ids_ng = lax.broadcasted_iota(jnp.int32, (N, G), 1)
    oh_ref[...] = (bcol_ref[...] == ids_ng).astype(_F32)
    ids_gn = lax.broadcasted_iota(jnp.int32, (G, N), 0)
    oht_ref[...] = (brow_ref[0:1, :] == ids_gn).astype(_F32)


_enc_kernel = pl.pallas_call(
    _enc_body,
    out_shape=[
        jax.ShapeDtypeStruct((N, D), _F32),
        jax.ShapeDtypeStruct((N, G), _F32),
        jax.ShapeDtypeStruct((G, N), _F32),
    ],
)

_EE_BLK = 4096


def _ee_body(ea_ref, w_ref, b_ref, out_ref):
    out_ref[...] = _dot(ea_ref[...], w_ref[...]) + b_ref[...]


_ee_kernel = pl.pallas_call(
    _ee_body,
    grid=(EPAD // _EE_BLK,),
    in_specs=[
        pl.BlockSpec((_EE_BLK, De), lambda i: (i, 0)),
        pl.BlockSpec((De, D), lambda i: (0, 0)),
        pl.BlockSpec((1, D), lambda i: (0, 0)),
    ],
    out_specs=pl.BlockSpec((_EE_BLK, D), lambda i: (i, 0)),
    out_shape=jax.ShapeDtypeStruct((EPAD, D), _F32),
)


def _node_core(hin_ref, agg_ref, scal_ref, w1_ref, b1_ref, g1_ref, be1_ref,
               w2_ref, b2_ref, bng_ref, bnb_ref, s1_ref, s2_ref):
    agg = agg_ref[0:N, :] + agg_ref[NP:NP + N, :]
    z = scal_ref[0, 0] * hin_ref[...] + agg
    s1_ref[...] = _dot(z, w1_ref[...]) + b1_ref[...]
    z = jnp.maximum(_bn_ref(s1_ref, N, g1_ref[...], be1_ref[...]), 0.0)
    s2_ref[...] = _dot(z, w2_ref[...]) + b2_ref[...]
    return _bn_ref(s2_ref, N, bng_ref[...], bnb_ref[...])


def _node_body(hin_ref, agg_ref, scal_ref, w1_ref, b1_ref, g1_ref, be1_ref,
               w2_ref, b2_ref, bng_ref, bnb_ref, oh_ref, oht_ref, vn_ref,
               vw1_ref, vb1_ref, vg1_ref, vbe1_ref, vw2_ref, vb2_ref,
               vg2_ref, vbe2_ref, hout_ref, vnout_ref, s1_ref, s2_ref):
    z = _node_core(hin_ref, agg_ref, scal_ref, w1_ref, b1_ref, g1_ref,
                   be1_ref, w2_ref, b2_ref, bng_ref, bnb_ref, s1_ref, s2_ref)
    h = jnp.maximum(z, 0.0)
    vt = (
        jnp.dot(oht_ref[...], hin_ref[...], preferred_element_type=_F32, precision=lax.Precision.HIGHEST)
        + vn_ref[...]
    )
    vt = _dot(vt, vw1_ref[...]) + vb1_ref[...]
    vt = jnp.maximum(_bn_small(vt, G, vg1_ref[...], vbe1_ref[...]), 0.0)
    vt = _dot(vt, vw2_ref[...]) + vb2_ref[...]
    vt = jnp.maximum(_bn_small(vt, G, vg2_ref[...], vbe2_ref[...]), 0.0)
    vnout_ref[...] = vt
    hout_ref[...] = h + jnp.dot(oh_ref[...], vt, preferred_element_type=_F32, precision=lax.Precision.HIGHEST)


_node_kernel = pl.pallas_call(
    _node_body,
    out_shape=[
        jax.ShapeDtypeStruct((N, D), _F32),
        jax.ShapeDtypeStruct((G, D), _F32),
    ],
    scratch_shapes=[
        pltpu.VMEM((N, 2 * D), _F32),
        pltpu.VMEM((N, D), _F32),
    ],
)


def _final_body(hin_ref, agg_ref, scal_ref, w1_ref, b1_ref, g1_ref, be1_ref,
                w2_ref, b2_ref, bng_ref, bnb_ref, oht_ref, hw_ref, hb_ref,
                out_ref, s1_ref, s2_ref):
    z = _node_core(hin_ref, agg_ref, scal_ref, w1_ref, b1_ref, g1_ref,
                   be1_ref, w2_ref, b2_ref, bng_ref, bnb_ref, s1_ref, s2_ref)
    cnt = jnp.dot(oht_ref[...], jnp.ones((N, 1), _F32),
                  preferred_element_type=_F32, precision=lax.Precision.HIGHEST)
    hg = jnp.dot(oht_ref[...], z, preferred_element_type=_F32, precision=lax.Precision.HIGHEST)
    hg = hg / jnp.maximum(cnt, 1.0)
    out_ref[...] = (
        _dot(hg, hw_ref[...]) + hb_ref[...]
    )


_final_kernel = pl.pallas_call(
    _final_body,
    out_shape=jax.ShapeDtypeStruct((G, T), _F32),
    scratch_shapes=[
        pltpu.VMEM((N, 2 * D), _F32),
        pltpu.VMEM((N, D), _F32),
    ],
)


def kernel(x, edge_attr, enc_W, enc_b, bond_W, bond_b, eps, mlp_W1, mlp_b1,
           mlp_g1, mlp_be1, mlp_W2, mlp_b2, bn_g, bn_b, vn_W1, vn_b1, vn_g1,
           vn_be1, vn_W2, vn_b2, vn_g2, vn_be2, head_W, head_b, edge_index,
           batch):
    pad = EPAD - E
    src = edge_index[0].astype(jnp.int32)
    dst = edge_index[1].astype(jnp.int32)
    perm = jnp.argsort(dst, stable=True)
    src = src[perm]
    dst = dst[perm]
    ea_s = edge_attr[perm]
    src_p = jnp.concatenate([src, jnp.zeros((pad,), jnp.int32)]).reshape(
        IDX_ROWS, CH)
    dst_p = jnp.concatenate(
        [dst, N + (jnp.arange(pad, dtype=jnp.int32) % (NP - N))]
    ).reshape(IDX_ROWS, CH)
    ea_p = jnp.pad(ea_s, ((0, pad), (0, 0)))

    b32 = batch.astype(jnp.int32)
    bcol = b32.reshape(N, 1)
    brow = jnp.broadcast_to(b32.reshape(1, N), (8, N))

    h_in, oh, oht = _enc_kernel(x, enc_W, enc_b.reshape(1, D), bcol, brow)

    vn = jnp.zeros((G, D), _F32)
    for l in range(L):
        ee = _ee_kernel(ea_p, bond_W[l], bond_b[l].reshape(1, D))
        agg2 = _edge_kernel(h_in, ee, src_p, dst_p)
        scal = (1.0 + eps[l]).reshape(1, 1)
        mlp_args = (scal, mlp_W1[l], mlp_b1[l].reshape(1, 2 * D),
                    mlp_g1[l].reshape(1, 2 * D), mlp_be1[l].reshape(1, 2 * D),
                    mlp_W2[l], mlp_b2[l].reshape(1, D),
                    bn_g[l].reshape(1, D), bn_b[l].reshape(1, D))
        if l < L - 1:
            h_in, vn = _node_kernel(
                h_in, agg2, *mlp_args, oh, oht, vn,
                vn_W1[l], vn_b1[l].reshape(1, 2 * D),
                vn_g1[l].reshape(1, 2 * D), vn_be1[l].reshape(1, 2 * D),
                vn_W2[l], vn_b2[l].reshape(1, D),
                vn_g2[l].reshape(1, D), vn_be2[l].reshape(1, D))
        else:
            out = _final_kernel(h_in, agg2, *mlp_args, oht, head_W,
                                head_b.reshape(1, T))
    return out


# speedup vs baseline: 1.6388x; 1.0769x over previous
import functools

import jax
import jax.numpy as jnp
from jax import lax
from jax.experimental import pallas as pl
from jax.experimental.pallas import tpu as pltpu
from jax.experimental.pallas import tpu_sc as plsc

N = 10000
E = 320000
D = 128
De = 16
L = 5
G = 64
T = 128

NC = 2
NS = 16
NW = NC * NS
CH = 128
CPT = 80
EPAD = NW * CPT * CH
IDX_ROWS = EPAD // CH
NP = N + 112
RPT = NP // NS
IHALF = CPT // 2

_F32 = jnp.float32


def _edge_body(hin, ee, srcr, dstr, out, acc, src_v, dst_v, ee_v, h_v, sem):
    c = lax.axis_index("c")
    s = lax.axis_index("s")
    wid = s * NC + c

    def _zrow(i, _):
        for dd in range(D // 16):
            h_v[i, pl.ds(dd * 16, 16)] = jnp.zeros((16,), _F32)
        return 0

    lax.fori_loop(0, CH, _zrow, 0)
    r0 = s * RPT
    off = 0
    while off < RPT:
        nrows = min(CH, RPT - off)
        pltpu.sync_copy(h_v.at[pl.ds(0, nrows)], acc.at[pl.ds(r0 + off, nrows)])
        off += nrows
    plsc.subcore_barrier()

    for half in range(CPT // IHALF):
        ibase = wid * CPT + half * IHALF
        pltpu.sync_copy(srcr.at[pl.ds(ibase, IHALF)], src_v)
        pltpu.sync_copy(dstr.at[pl.ds(ibase, IHALF)], dst_v)

        def _chunk(j, _):
            base = (ibase + j) * CH
            pltpu.sync_copy(ee.at[pl.ds(base, CH)], ee_v)
            pltpu.async_copy(hin.at[src_v.at[j]], h_v, sem).wait()

            def _crow(i, _):
                for dd in range(D // 16):
                    sl = pl.ds(dd * 16, 16)
                    h_v[i, sl] = jnp.maximum(h_v[i, sl] + ee_v[i, sl], 0.0)
                return 0

            lax.fori_loop(0, CH, _crow, 0)
            pltpu.sync_copy(h_v, acc.at[dst_v.at[j]], add=True)
            return 0

        lax.fori_loop(0, IHALF, _chunk, 0)
    plsc.subcore_barrier()

    pltpu.sync_copy(acc.at[pl.ds(s * RPT, RPT)],
                    out.at[pl.ds(c * NP + s * RPT, RPT)])


_edge_kernel = functools.partial(
    pl.kernel,
    out_type=jax.ShapeDtypeStruct((2 * NP, D), _F32),
    mesh=plsc.VectorSubcoreMesh(core_axis_name="c", subcore_axis_name="s"),
    scratch_types=[
        pltpu.VMEM_SHARED((NP, D), _F32),
        pltpu.VMEM((IHALF, CH), jnp.int32),
        pltpu.VMEM((IHALF, CH), jnp.int32),
        pltpu.VMEM((CH, D), _F32),
        pltpu.VMEM((CH, D), _F32),
        pltpu.SemaphoreType.DMA,
    ],
)(_edge_body)


def _dot(a, b):
    return jnp.dot(a.astype(jnp.bfloat16), b.astype(jnp.bfloat16),
                   preferred_element_type=_F32)


def _bn_stats_ref(ref, n):
    t = n // 8
    bb = t // 2
    cc = ref.shape[1]

    def mstep(k, accs):
        a0, a1 = accs
        return (a0 + ref[pl.ds(8 * k, 8), :], a1 + ref[pl.ds(8 * (bb + k), 8), :])

    a0, a1 = lax.fori_loop(0, bb, mstep,
                           (jnp.zeros((8, cc), _F32), jnp.zeros((8, cc), _F32)))
    m = jnp.sum(a0 + a1, axis=0, keepdims=True) / float(n)

    def vstep(k, accs):
        a0, a1 = accs
        d0 = ref[pl.ds(8 * k, 8), :] - m
        d1 = ref[pl.ds(8 * (bb + k), 8), :] - m
        return (a0 + d0 * d0, a1 + d1 * d1)

    a0, a1 = lax.fori_loop(0, bb, vstep,
                           (jnp.zeros((8, cc), _F32), jnp.zeros((8, cc), _F32)))
    v = jnp.sum(a0 + a1, axis=0, keepdims=True) / float(n)
    return m, v


def _bn_ref(ref, n, g, b):
    m, v = _bn_stats_ref(ref, n)
    return g * (ref[...] - m) / jnp.sqrt(v + 1e-5) + b


def _bn_small(h, n, g, b):
    m = jnp.mean(h, axis=0, keepdims=True)
    v = jnp.mean((h - m) ** 2, axis=0, keepdims=True)
    return g * (h - m) / jnp.sqrt(v + 1e-5) + b


def _enc_body(x_ref, w_ref, b_ref, bcol_ref, brow_ref, h_ref, oh_ref, oht_ref):
    h_ref[...] = _dot(x_ref[...], w_ref[...]) + b_ref[...]
    ids_ng = lax.broadcasted_iota(jnp.int32, (N, G), 1)
    oh_ref[...] = (bcol_ref[...] == ids_ng).astype(_F32)
    ids_gn = lax.broadcasted_iota(jnp.int32, (G, N), 0)
    oht_ref[...] = (brow_ref[0:1, :] == ids_gn).astype(_F32)


_enc_kernel = pl.pallas_call(
    _enc_body,
    out_shape=[
        jax.ShapeDtypeStruct((N, D), _F32),
        jax.ShapeDtypeStruct((N, G), _F32),
        jax.ShapeDtypeStruct((G, N), _F32),
    ],
)

_EE_BLK = 4096


def _ee_body(ea_ref, w_ref, b_ref, out_ref):
    out_ref[...] = _dot(ea_ref[...], w_ref[...]) + b_ref[...]


_ee_kernel = pl.pallas_call(
    _ee_body,
    grid=(EPAD // _EE_BLK,),
    in_specs=[
        pl.BlockSpec((_EE_BLK, De), lambda i: (i, 0)),
        pl.BlockSpec((De, D), lambda i: (0, 0)),
        pl.BlockSpec((1, D), lambda i: (0, 0)),
    ],
    out_specs=pl.BlockSpec((_EE_BLK, D), lambda i: (i, 0)),
    out_shape=jax.ShapeDtypeStruct((EPAD, D), _F32),
)


def _node_core(hin_ref, agg_ref, scal_ref, w1_ref, b1_ref, g1_ref, be1_ref,
               w2_ref, b2_ref, bng_ref, bnb_ref, s1_ref, s2_ref):
    agg = agg_ref[0:N, :] + agg_ref[NP:NP + N, :]
    z = scal_ref[0, 0] * hin_ref[...] + agg
    s1_ref[...] = _dot(z, w1_ref[...]) + b1_ref[...]
    z = jnp.maximum(_bn_ref(s1_ref, N, g1_ref[...], be1_ref[...]), 0.0)
    s2_ref[...] = _dot(z, w2_ref[...]) + b2_ref[...]
    return _bn_ref(s2_ref, N, bng_ref[...], bnb_ref[...])


def _node_body(hin_ref, agg_ref, scal_ref, w1_ref, b1_ref, g1_ref, be1_ref,
               w2_ref, b2_ref, bng_ref, bnb_ref, oh_ref, oht_ref, vn_ref,
               vw1_ref, vb1_ref, vg1_ref, vbe1_ref, vw2_ref, vb2_ref,
               vg2_ref, vbe2_ref, hout_ref, vnout_ref, s1_ref, s2_ref):
    z = _node_core(hin_ref, agg_ref, scal_ref, w1_ref, b1_ref, g1_ref,
                   be1_ref, w2_ref, b2_ref, bng_ref, bnb_ref, s1_ref, s2_ref)
    h = jnp.maximum(z, 0.0)
    vt = (
        jnp.dot(oht_ref[...], hin_ref[...], preferred_element_type=_F32, precision=lax.Precision.HIGHEST)
        + vn_ref[...]
    )
    vt = _dot(vt, vw1_ref[...]) + vb1_ref[...]
    vt = jnp.maximum(_bn_small(vt, G, vg1_ref[...], vbe1_ref[...]), 0.0)
    vt = _dot(vt, vw2_ref[...]) + vb2_ref[...]
    vt = jnp.maximum(_bn_small(vt, G, vg2_ref[...], vbe2_ref[...]), 0.0)
    vnout_ref[...] = vt
    hout_ref[...] = h + jnp.dot(oh_ref[...], vt, preferred_element_type=_F32, precision=lax.Precision.HIGHEST)


_node_kernel = pl.pallas_call(
    _node_body,
    out_shape=[
        jax.ShapeDtypeStruct((N, D), _F32),
        jax.ShapeDtypeStruct((G, D), _F32),
    ],
    scratch_shapes=[
        pltpu.VMEM((N, 2 * D), _F32),
        pltpu.VMEM((N, D), _F32),
    ],
)


def _final_body(hin_ref, agg_ref, scal_ref, w1_ref, b1_ref, g1_ref, be1_ref,
                w2_ref, b2_ref, bng_ref, bnb_ref, oht_ref, hw_ref, hb_ref,
                out_ref, s1_ref, s2_ref):
    z = _node_core(hin_ref, agg_ref, scal_ref, w1_ref, b1_ref, g1_ref,
                   be1_ref, w2_ref, b2_ref, bng_ref, bnb_ref, s1_ref, s2_ref)
    cnt = jnp.dot(oht_ref[...], jnp.ones((N, 1), _F32),
                  preferred_element_type=_F32, precision=lax.Precision.HIGHEST)
    hg = jnp.dot(oht_ref[...], z, preferred_element_type=_F32, precision=lax.Precision.HIGHEST)
    hg = hg / jnp.maximum(cnt, 1.0)
    out_ref[...] = (
        _dot(hg, hw_ref[...]) + hb_ref[...]
    )


_final_kernel = pl.pallas_call(
    _final_body,
    out_shape=jax.ShapeDtypeStruct((G, T), _F32),
    scratch_shapes=[
        pltpu.VMEM((N, 2 * D), _F32),
        pltpu.VMEM((N, D), _F32),
    ],
)


def kernel(x, edge_attr, enc_W, enc_b, bond_W, bond_b, eps, mlp_W1, mlp_b1,
           mlp_g1, mlp_be1, mlp_W2, mlp_b2, bn_g, bn_b, vn_W1, vn_b1, vn_g1,
           vn_be1, vn_W2, vn_b2, vn_g2, vn_be2, head_W, head_b, edge_index,
           batch):
    pad = EPAD - E
    src = edge_index[0].astype(jnp.int32)
    dst = edge_index[1].astype(jnp.int32)
    perm = jnp.argsort(dst, stable=True)
    src = src[perm]
    dst = dst[perm]
    ea_s = edge_attr[perm]
    src_p = jnp.concatenate([src, jnp.zeros((pad,), jnp.int32)]).reshape(
        IDX_ROWS, CH)
    dst_p = jnp.concatenate(
        [dst, N + (jnp.arange(pad, dtype=jnp.int32) % (NP - N))]
    ).reshape(IDX_ROWS, CH)
    ea_p = jnp.pad(ea_s, ((0, pad), (0, 0)))

    b32 = batch.astype(jnp.int32)
    bcol = b32.reshape(N, 1)
    brow = jnp.broadcast_to(b32.reshape(1, N), (8, N))

    h_in, oh, oht = _enc_kernel(x, enc_W, enc_b.reshape(1, D), bcol, brow)

    vn = jnp.zeros((G, D), _F32)
    for l in range(L):
        ee = _ee_kernel(ea_p, bond_W[l], bond_b[l].reshape(1, D))
        agg2 = _edge_kernel(h_in, ee, src_p, dst_p)
        scal = (1.0 + eps[l]).reshape(1, 1)
        mlp_args = (scal, mlp_W1[l], mlp_b1[l].reshape(1, 2 * D),
                    mlp_g1[l].reshape(1, 2 * D), mlp_be1[l].reshape(1, 2 * D),
                    mlp_W2[l], mlp_b2[l].reshape(1, D),
                    bn_g[l].reshape(1, D), bn_b[l].reshape(1, D))
        if l < L - 1:
            h_in, vn = _node_kernel(
                h_in, agg2, *mlp_args, oh, oht, vn,
                vn_W1[l], vn_b1[l].reshape(1, 2 * D),
                vn_g1[l].reshape(1, 2 * D), vn_be1[l].reshape(1, 2 * D),
                vn_W2[l], vn_b2[l].reshape(1, D),
                vn_g2[l].reshape(1, D), vn_be2[l].reshape(1, D))
        else:
            out = _final_kernel(h_in, agg2, *mlp_args, oht, head_W,
                                head_b.reshape(1, T))
    return out
